# baseline (speedup 1.0000x reference)
import functools

import jax
import jax.numpy as jnp
from jax import lax
from jax.experimental import pallas as pl
from jax.experimental.pallas import tpu as pltpu
from jax.experimental.pallas import tpu_sc as plsc

_VOCAB = 100000
_EMB = 128
_HID = 768
_MAXPOS = 4096
_TYPES = 4
_B, _S = 4, 4096
_TOK = _B * _S
_EPS = 1e-12

_NHALF = 2
_HTOK = _TOK // _NHALF
_HB = _B // _NHALF


_NC, _NS = 2, 16
_NW = _NC * _NS
_ROWS_PER_W = _HTOK // _NW
_CHUNK = 128
_NCHUNK = _ROWS_PER_W // _CHUNK


def _sc_gather_body(half, table_hbm, idx_hbm, out_hbm, idx_v, rows_v, gsem, wsem):
    wid = lax.axis_index("s") * _NC + lax.axis_index("c")
    idx_row0 = half * (_HTOK // _CHUNK) + wid * _NCHUNK
    pltpu.sync_copy(idx_hbm.at[pl.ds(idx_row0, _NCHUNK)], idx_v)
    gathers = []
    for j in range(_NCHUNK):
        gathers.append(
            pltpu.async_copy(
                table_hbm.at[idx_v.at[j]],
                rows_v.at[pl.ds(j * _CHUNK, _CHUNK)],
                gsem,
            )
        )
    writes = []
    for j in range(_NCHUNK):
        gathers[j].wait()
        writes.append(
            pltpu.async_copy(
                rows_v.at[pl.ds(j * _CHUNK, _CHUNK)],
                out_hbm.at[pl.ds(wid * _ROWS_PER_W + j * _CHUNK, _CHUNK)],
                wsem,
            )
        )
    for cp in writes:
        cp.wait()


@functools.cache
def _sc_gather(half):
    return functools.partial(
        pl.kernel,
        mesh=plsc.VectorSubcoreMesh(core_axis_name="c", subcore_axis_name="s"),
        out_type=jax.ShapeDtypeStruct((_HTOK, _EMB), jnp.float32),
        scratch_types=[
            pltpu.VMEM((_NCHUNK, _CHUNK), jnp.int32),
            pltpu.VMEM((_ROWS_PER_W, _EMB), jnp.float32),
            pltpu.SemaphoreType.DMA,
            pltpu.SemaphoreType.DMA,
        ],
    )(functools.partial(_sc_gather_body, half))


_BT = 1024
_JB = _MAXPOS // _BT


def _tc_body(g_ref, p_ref, t_ref, te_ref, gam_ref, bet_ref, w_ref, b_ref,
             *prev_and_out):
    o_ref = prev_and_out[-1]
    tid = t_ref[...]
    oh = (
        lax.broadcasted_iota(jnp.int32, (8, _BT), 0) == tid
    ).astype(jnp.float32)
    te = lax.dot_general(
        oh, te_ref[...], (((0,), (0,)), ((), ())),
        preferred_element_type=jnp.float32,
    )
    acc = g_ref[...] + p_ref[...] + te
    mu = jnp.mean(acc, axis=1, keepdims=True)
    d = acc - mu
    var = jnp.mean(d * d, axis=1, keepdims=True)
    y = d * lax.rsqrt(var + _EPS) * gam_ref[...] + bet_ref[...]
    o_ref[...] = (
        jnp.dot(y, w_ref[...], preferred_element_type=jnp.float32) + b_ref[...]
    )


@functools.cache
def _tc_fused(half):
    def row(j, bi):
        return (half * _HB + bi) * _JB + j

    specs = [
        pl.BlockSpec((_BT, _EMB), lambda j, bi: (bi * _JB + j, 0)),
        pl.BlockSpec((_BT, _EMB), lambda j, bi: (j, 0)),
        pl.BlockSpec((1, _BT), lambda j, bi: (0, row(j, bi))),
        pl.BlockSpec((8, _EMB), lambda j, bi: (0, 0)),
        pl.BlockSpec((1, _EMB), lambda j, bi: (0, 0)),
        pl.BlockSpec((1, _EMB), lambda j, bi: (0, 0)),
        pl.BlockSpec((_EMB, _HID), lambda j, bi: (0, 0)),
        pl.BlockSpec((1, _HID), lambda j, bi: (0, 0)),
    ]
    aliases = {}
    if half > 0:
        specs.append(pl.BlockSpec(memory_space=pl.ANY))
        aliases = {8: 0}
    return pl.pallas_call(
        _tc_body,
        grid=(_JB, _HB),
        in_specs=specs,
        out_specs=pl.BlockSpec((_BT, _HID), lambda j, bi: (row(j, bi), 0)),
        out_shape=jax.ShapeDtypeStruct((_TOK, _HID), jnp.float32),
        input_output_aliases=aliases,
    )


def kernel(input_ids, type_ids, word_emb, pos_emb, type_emb, gamma, beta, W, b):
    ids2d = input_ids.astype(jnp.int32).reshape(_TOK // _CHUNK, _CHUNK)
    te_pad = jnp.zeros((8, _EMB), jnp.float32).at[:_TYPES].set(type_emb)
    tid_row = type_ids.astype(jnp.int32).reshape(1, _TOK)
    gamma2 = gamma.reshape(1, _EMB)
    beta2 = beta.reshape(1, _EMB)
    b2 = b.reshape(1, _HID)

    gath = [_sc_gather(h)(word_emb, ids2d) for h in range(_NHALF)]
    out = _tc_fused(0)(gath[0], pos_emb, tid_row, te_pad, gamma2, beta2, W, b2)
    for h in range(1, _NHALF):
        out = _tc_fused(h)(
            gath[h], pos_emb, tid_row, te_pad, gamma2, beta2, W, b2, out
        )
    return out.reshape(_B, _S, _HID)


# speedup vs baseline: 1.1592x; 1.1592x over previous
import functools

import jax
import jax.numpy as jnp
from jax import lax
from jax.experimental import pallas as pl
from jax.experimental.pallas import tpu as pltpu
from jax.experimental.pallas import tpu_sc as plsc

_VOCAB = 100000
_EMB = 128
_HID = 768
_MAXPOS = 4096
_TYPES = 4
_B, _S = 4, 4096
_TOK = _B * _S
_EPS = 1e-12

_NHALF = 1
_HTOK = _TOK // _NHALF
_HB = _B // _NHALF


_NC, _NS = 2, 16
_NW = _NC * _NS
_ROWS_PER_W = _HTOK // _NW
_CHUNK = 128
_NCHUNK = _ROWS_PER_W // _CHUNK


def _sc_gather_body(half, table_hbm, idx_hbm, out_hbm, idx_v, rows_v, gsem, wsem):
    wid = lax.axis_index("s") * _NC + lax.axis_index("c")
    idx_row0 = half * (_HTOK // _CHUNK) + wid * _NCHUNK
    pltpu.sync_copy(idx_hbm.at[pl.ds(idx_row0, _NCHUNK)], idx_v)
    gathers = []
    for j in range(_NCHUNK):
        gathers.append(
            pltpu.async_copy(
                table_hbm.at[idx_v.at[j]],
                rows_v.at[pl.ds(j * _CHUNK, _CHUNK)],
                gsem,
            )
        )
    writes = []
    for j in range(_NCHUNK):
        gathers[j].wait()
        writes.append(
            pltpu.async_copy(
                rows_v.at[pl.ds(j * _CHUNK, _CHUNK)],
                out_hbm.at[pl.ds(wid * _ROWS_PER_W + j * _CHUNK, _CHUNK)],
                wsem,
            )
        )
    for cp in writes:
        cp.wait()


@functools.cache
def _sc_gather(half):
    return functools.partial(
        pl.kernel,
        mesh=plsc.VectorSubcoreMesh(core_axis_name="c", subcore_axis_name="s"),
        out_type=jax.ShapeDtypeStruct((_HTOK, _EMB), jnp.float32),
        scratch_types=[
            pltpu.VMEM((_NCHUNK, _CHUNK), jnp.int32),
            pltpu.VMEM((_ROWS_PER_W, _EMB), jnp.float32),
            pltpu.SemaphoreType.DMA,
            pltpu.SemaphoreType.DMA,
        ],
    )(functools.partial(_sc_gather_body, half))


_BT = 2048
_JB = _MAXPOS // _BT


def _tc_body(g_ref, p_ref, t_ref, te_ref, gam_ref, bet_ref, w_ref, b_ref,
             *prev_and_out):
    o_ref = prev_and_out[-1]
    tid = t_ref[...]
    oh = (
        lax.broadcasted_iota(jnp.int32, (8, _BT), 0) == tid
    ).astype(jnp.float32)
    te = lax.dot_general(
        oh, te_ref[...], (((0,), (0,)), ((), ())),
        preferred_element_type=jnp.float32,
    )
    acc = g_ref[...] + p_ref[...] + te
    mu = jnp.mean(acc, axis=1, keepdims=True)
    d = acc - mu
    var = jnp.mean(d * d, axis=1, keepdims=True)
    y = d * lax.rsqrt(var + _EPS) * gam_ref[...] + bet_ref[...]
    o_ref[...] = (
        jnp.dot(y, w_ref[...], preferred_element_type=jnp.float32) + b_ref[...]
    )


@functools.cache
def _tc_fused(half):
    def row(j, bi):
        return (half * _HB + bi) * _JB + j

    specs = [
        pl.BlockSpec((_BT, _EMB), lambda j, bi: (bi * _JB + j, 0)),
        pl.BlockSpec((_BT, _EMB), lambda j, bi: (j, 0)),
        pl.BlockSpec((1, _BT), lambda j, bi: (0, row(j, bi))),
        pl.BlockSpec((8, _EMB), lambda j, bi: (0, 0)),
        pl.BlockSpec((1, _EMB), lambda j, bi: (0, 0)),
        pl.BlockSpec((1, _EMB), lambda j, bi: (0, 0)),
        pl.BlockSpec((_EMB, _HID), lambda j, bi: (0, 0)),
        pl.BlockSpec((1, _HID), lambda j, bi: (0, 0)),
    ]
    aliases = {}
    if half > 0:
        specs.append(pl.BlockSpec(memory_space=pl.ANY))
        aliases = {8: 0}
    return pl.pallas_call(
        _tc_body,
        grid=(_JB, _HB),
        in_specs=specs,
        out_specs=pl.BlockSpec((_BT, _HID), lambda j, bi: (row(j, bi), 0)),
        out_shape=jax.ShapeDtypeStruct((_TOK, _HID), jnp.float32),
        input_output_aliases=aliases,
    )


def kernel(input_ids, type_ids, word_emb, pos_emb, type_emb, gamma, beta, W, b):
    ids2d = input_ids.astype(jnp.int32).reshape(_TOK // _CHUNK, _CHUNK)
    te_pad = jnp.zeros((8, _EMB), jnp.float32).at[:_TYPES].set(type_emb)
    tid_row = type_ids.astype(jnp.int32).reshape(1, _TOK)
    gamma2 = gamma.reshape(1, _EMB)
    beta2 = beta.reshape(1, _EMB)
    b2 = b.reshape(1, _HID)

    gath = [_sc_gather(h)(word_emb, ids2d) for h in range(_NHALF)]
    out = _tc_fused(0)(gath[0], pos_emb, tid_row, te_pad, gamma2, beta2, W, b2)
    for h in range(1, _NHALF):
        out = _tc_fused(h)(
            gath[h], pos_emb, tid_row, te_pad, gamma2, beta2, W, b2, out
        )
    return out.reshape(_B, _S, _HID)


# speedup vs baseline: 1.1593x; 1.0001x over previous
import functools

import jax
import jax.numpy as jnp
from jax import lax
from jax.experimental import pallas as pl
from jax.experimental.pallas import tpu as pltpu
from jax.experimental.pallas import tpu_sc as plsc

_VOCAB = 100000
_EMB = 128
_HID = 768
_MAXPOS = 4096
_TYPES = 4
_B, _S = 4, 4096
_TOK = _B * _S
_EPS = 1e-12

_NHALF = 1
_HTOK = _TOK // _NHALF
_HB = _B // _NHALF


_NC, _NS = 2, 16
_NW = _NC * _NS
_ROWS_PER_W = _HTOK // _NW
_CHUNK = 128
_NCHUNK = _ROWS_PER_W // _CHUNK


def _sc_gather_body(half, table_hbm, idx_hbm, out_hbm, idx_v, rows_v, gsem, wsem):
    wid = lax.axis_index("s") * _NC + lax.axis_index("c")
    idx_row0 = half * (_HTOK // _CHUNK) + wid * _NCHUNK
    pltpu.sync_copy(idx_hbm.at[pl.ds(idx_row0, _NCHUNK)], idx_v)
    gathers = []
    for j in range(_NCHUNK):
        gathers.append(
            pltpu.async_copy(
                table_hbm.at[idx_v.at[j]],
                rows_v.at[pl.ds(j * _CHUNK, _CHUNK)],
                gsem,
            )
        )
    writes = []
    for j in range(_NCHUNK):
        gathers[j].wait()
        writes.append(
            pltpu.async_copy(
                rows_v.at[pl.ds(j * _CHUNK, _CHUNK)],
                out_hbm.at[pl.ds(wid * _ROWS_PER_W + j * _CHUNK, _CHUNK)],
                wsem,
            )
        )
    for cp in writes:
        cp.wait()


@functools.cache
def _sc_gather(half):
    return functools.partial(
        pl.kernel,
        mesh=plsc.VectorSubcoreMesh(core_axis_name="c", subcore_axis_name="s"),
        out_type=jax.ShapeDtypeStruct((_HTOK, _EMB), jnp.float32),
        scratch_types=[
            pltpu.VMEM((_NCHUNK, _CHUNK), jnp.int32),
            pltpu.VMEM((_ROWS_PER_W, _EMB), jnp.float32),
            pltpu.SemaphoreType.DMA,
            pltpu.SemaphoreType.DMA,
        ],
    )(functools.partial(_sc_gather_body, half))


_BT = 4096
_JB = _MAXPOS // _BT


def _tc_body(g_ref, p_ref, t_ref, te_ref, gam_ref, bet_ref, w_ref, b_ref,
             *prev_and_out):
    o_ref = prev_and_out[-1]
    tid = t_ref[...]
    oh = (
        lax.broadcasted_iota(jnp.int32, (8, _BT), 0) == tid
    ).astype(jnp.float32)
    te = lax.dot_general(
        oh, te_ref[...], (((0,), (0,)), ((), ())),
        preferred_element_type=jnp.float32,
    )
    acc = g_ref[...] + p_ref[...] + te
    mu = jnp.mean(acc, axis=1, keepdims=True)
    d = acc - mu
    var = jnp.mean(d * d, axis=1, keepdims=True)
    y = d * lax.rsqrt(var + _EPS) * gam_ref[...] + bet_ref[...]
    o_ref[...] = (
        jnp.dot(y, w_ref[...], preferred_element_type=jnp.float32) + b_ref[...]
    )


@functools.cache
def _tc_fused(half):
    def row(j, bi):
        return (half * _HB + bi) * _JB + j

    specs = [
        pl.BlockSpec((_BT, _EMB), lambda j, bi: (bi * _JB + j, 0)),
        pl.BlockSpec((_BT, _EMB), lambda j, bi: (j, 0)),
        pl.BlockSpec((1, _BT), lambda j, bi: (0, row(j, bi))),
        pl.BlockSpec((8, _EMB), lambda j, bi: (0, 0)),
        pl.BlockSpec((1, _EMB), lambda j, bi: (0, 0)),
        pl.BlockSpec((1, _EMB), lambda j, bi: (0, 0)),
        pl.BlockSpec((_EMB, _HID), lambda j, bi: (0, 0)),
        pl.BlockSpec((1, _HID), lambda j, bi: (0, 0)),
    ]
    aliases = {}
    if half > 0:
        specs.append(pl.BlockSpec(memory_space=pl.ANY))
        aliases = {8: 0}
    return pl.pallas_call(
        _tc_body,
        grid=(_JB, _HB),
        in_specs=specs,
        out_specs=pl.BlockSpec((_BT, _HID), lambda j, bi: (row(j, bi), 0)),
        out_shape=jax.ShapeDtypeStruct((_TOK, _HID), jnp.float32),
        input_output_aliases=aliases,
    )


def kernel(input_ids, type_ids, word_emb, pos_emb, type_emb, gamma, beta, W, b):
    ids2d = input_ids.astype(jnp.int32).reshape(_TOK // _CHUNK, _CHUNK)
    te_pad = jnp.zeros((8, _EMB), jnp.float32).at[:_TYPES].set(type_emb)
    tid_row = type_ids.astype(jnp.int32).reshape(1, _TOK)
    gamma2 = gamma.reshape(1, _EMB)
    beta2 = beta.reshape(1, _EMB)
    b2 = b.reshape(1, _HID)

    gath = [_sc_gather(h)(word_emb, ids2d) for h in range(_NHALF)]
    out = _tc_fused(0)(gath[0], pos_emb, tid_row, te_pad, gamma2, beta2, W, b2)
    for h in range(1, _NHALF):
        out = _tc_fused(h)(
            gath[h], pos_emb, tid_row, te_pad, gamma2, beta2, W, b2, out
        )
    return out.reshape(_B, _S, _HID)
